# Initial kernel scaffold; baseline (speedup 1.0000x reference)
import jax
import jax.numpy as jnp
from jax.experimental import pallas as pl


def kernel(scores, bbox_deltas, im_info):
    raise NotImplementedError("write your pallas kernel here")


# speedup vs baseline: 45.2495x; 45.2495x over previous
import numpy as np
import jax
import jax.numpy as jnp
from jax import lax
from jax.experimental import pallas as pl
from jax.experimental.pallas import tpu as pltpu

PRE_NMS_TOPN = 6000
POST_NMS_TOPN = 300
NMS_THRESH = 0.7
A = 9
H = 64
W = 64
K = H * W
N = A * K
OUT_ROWS = 304


def _anchors_np(base_size=16, ratios=(0.5, 1.0, 2.0), scales=(8, 16, 32)):
    ratios = np.asarray(ratios, dtype=np.float64)
    scales = np.asarray(scales, dtype=np.float64)
    base = np.array([1, 1, base_size, base_size], dtype=np.float64) - 1
    w = base[2] - base[0] + 1
    h = base[3] - base[1] + 1
    x_ctr = base[0] + 0.5 * (w - 1)
    y_ctr = base[1] + 0.5 * (h - 1)
    size = w * h
    ws = np.round(np.sqrt(size / ratios))
    hs = np.round(ws * ratios)
    ratio_anchors = np.stack(
        [x_ctr - 0.5 * (ws - 1), y_ctr - 0.5 * (hs - 1),
         x_ctr + 0.5 * (ws - 1), y_ctr + 0.5 * (hs - 1)], axis=1)
    out = []
    for a in ratio_anchors:
        w2 = a[2] - a[0] + 1
        h2 = a[3] - a[1] + 1
        xc = a[0] + 0.5 * (w2 - 1)
        yc = a[1] + 0.5 * (h2 - 1)
        ws2 = w2 * scales
        hs2 = h2 * scales
        out.append(np.stack(
            [xc - 0.5 * (ws2 - 1), yc - 0.5 * (hs2 - 1),
             xc + 0.5 * (ws2 - 1), yc + 0.5 * (hs2 - 1)], axis=1))
    return np.vstack(out).astype(np.float32)


_ANCH = _anchors_np()


def _favg(lo, hi):
    return (lo & hi) + ((lo ^ hi) >> 1)


def _proposal_kernel(scr_ref, dx_ref, dy_ref, dw_ref, dh_ref,
                     ax1_ref, ay1_ref, ax2_ref, ay2_ref, im_ref, out_ref,
                     alive_ref):
    ki = lax.broadcasted_iota(jnp.int32, (A, K), 1)
    ai = lax.broadcasted_iota(jnp.int32, (A, K), 0)
    sx = ((ki >> 6) << 4).astype(jnp.float32)
    sy = ((ki & 63) << 4).astype(jnp.float32)

    x1a = ax1_ref[...] + sx
    y1a = ay1_ref[...] + sy
    x2a = ax2_ref[...] + sx
    y2a = ay2_ref[...] + sy
    widths = x2a - x1a + 1.0
    heights = y2a - y1a + 1.0
    ctr_x = x1a + 0.5 * widths
    ctr_y = y1a + 0.5 * heights

    dx = dx_ref[...]
    dy = dy_ref[...]
    dw = dw_ref[...]
    dh = dh_ref[...]
    pcx = dx * widths + ctr_x
    pcy = dy * heights + ctr_y
    pw = jnp.exp(dw) * widths
    ph = jnp.exp(dh) * heights

    im0 = im_ref[0]
    im1 = im_ref[1]
    im2 = im_ref[2]
    zero = jnp.float32(0.0)
    x1 = jnp.maximum(jnp.minimum(pcx - 0.5 * pw, im1 - 1), zero)
    y1 = jnp.maximum(jnp.minimum(pcy - 0.5 * ph, im0 - 1), zero)
    x2 = jnp.maximum(jnp.minimum(pcx + 0.5 * pw, im1 - 1), zero)
    y2 = jnp.maximum(jnp.minimum(pcy + 0.5 * ph, im0 - 1), zero)

    ws_ = x2 - x1 + 1.0
    hs_ = y2 - y1 + 1.0
    min_sz = 0.0 * im2
    valid = (ws_ >= min_sz) & (hs_ >= min_sz)
    scrv = jnp.where(valid, scr_ref[...], -jnp.inf)

    b = lax.bitcast_convert_type(scrv, jnp.int32)
    key = b ^ (jnp.right_shift(b, 31) & jnp.int32(0x7FFFFFFF))
    idxn = ki * A + ai
    area = (x2 - x1) * (y2 - y1)

    def bs1(_, c):
        lo, hi = c
        mid = _favg(lo, hi)
        cnt = jnp.sum((key >= mid).astype(jnp.int32))
        p = cnt < PRE_NMS_TOPN
        return (jnp.where(p, lo, mid), jnp.where(p, mid, hi))

    lo, hi = lax.fori_loop(
        0, 32, bs1, (jnp.int32(-2**31), jnp.int32(2**31 - 1)))
    v_thr = hi - 1

    cnt_gt = jnp.sum((key > v_thr).astype(jnp.int32))
    need_eq = PRE_NMS_TOPN - cnt_gt
    eq = key == v_thr

    def bs2(_, c):
        lo, hi = c
        mid = _favg(lo, hi)
        cnt = jnp.sum((eq & (idxn <= mid)).astype(jnp.int32))
        q = cnt >= need_eq
        return (jnp.where(q, lo, mid), jnp.where(q, mid, hi))

    _, nstar = lax.fori_loop(0, 17, bs2, (jnp.int32(-1), jnp.int32(N - 1)))
    sel = (key > v_thr) | (eq & (idxn <= nstar))

    int_min = jnp.int32(-2**31)
    big = jnp.int32(2**31 - 1)
    lane = lax.broadcasted_iota(jnp.int32, (1, 128), 1)
    thresh = jnp.float32(NMS_THRESH)
    eps = jnp.float32(1e-12)

    alive_ref[...] = sel.astype(jnp.int32)

    def body(t, c):
        fx1, fy1, fx2, fy2 = c
        alive = alive_ref[...] != 0
        mkey = jnp.max(jnp.where(alive, key, int_min))
        has = mkey > int_min
        cm = alive & (key == mkey)
        mn = jnp.min(jnp.where(cm, idxn, big))
        m1 = cm & (idxn == mn)
        m1f = m1.astype(jnp.float32)
        bx1 = jnp.sum(x1 * m1f)
        by1 = jnp.sum(y1 * m1f)
        bx2 = jnp.sum(x2 * m1f)
        by2 = jnp.sum(y2 * m1f)
        bx1 = jnp.where(has, bx1, fx1)
        by1 = jnp.where(has, by1, fy1)
        bx2 = jnp.where(has, bx2, fx2)
        by2 = jnp.where(has, by2, fy2)
        is0 = t == 0
        nfx1 = jnp.where(is0, bx1, fx1)
        nfy1 = jnp.where(is0, by1, fy1)
        nfx2 = jnp.where(is0, bx2, fx2)
        nfy2 = jnp.where(is0, by2, fy2)

        xx1 = jnp.maximum(x1, bx1)
        yy1 = jnp.maximum(y1, by1)
        xx2 = jnp.minimum(x2, bx2)
        yy2 = jnp.minimum(y2, by2)
        inter = jnp.clip(xx2 - xx1, 0.0) * jnp.clip(yy2 - yy1, 0.0)
        barea = (bx2 - bx1) * (by2 - by1)
        iou = inter / (barea + area - inter + eps)
        supp = ((iou > thresh) | m1) & has
        alive_ref[...] = (alive & (~supp)).astype(jnp.int32)

        row = jnp.where(lane == 0, bx1,
              jnp.where(lane == 1, by1,
              jnp.where(lane == 2, bx2,
              jnp.where(lane == 3, by2, zero))))
        out_ref[pl.ds(t, 1), :] = row
        return (nfx1, nfy1, nfx2, nfy2)

    lax.fori_loop(0, POST_NMS_TOPN, body, (zero, zero, zero, zero))


def kernel(scores, bbox_deltas, im_info):
    scr = scores.reshape(2, A, K)[1]
    d = bbox_deltas.reshape(A, 4, K)
    dx = d[:, 0, :]
    dy = d[:, 1, :]
    dw = d[:, 2, :]
    dh = d[:, 3, :]
    anch = jnp.asarray(_ANCH)
    ax1 = anch[:, 0:1]
    ay1 = anch[:, 1:2]
    ax2 = anch[:, 2:3]
    ay2 = anch[:, 3:4]

    vspec = pl.BlockSpec(memory_space=pltpu.VMEM)
    buf = pl.pallas_call(
        _proposal_kernel,
        out_shape=jax.ShapeDtypeStruct((OUT_ROWS, 128), jnp.float32),
        in_specs=[vspec] * 9 + [pl.BlockSpec(memory_space=pltpu.SMEM)],
        out_specs=pl.BlockSpec(memory_space=pltpu.VMEM),
        scratch_shapes=[pltpu.VMEM((A, K), jnp.int32)],
    )(scr, dx, dy, dw, dh, ax1, ay1, ax2, ay2, im_info)

    zeros = jnp.zeros((POST_NMS_TOPN, 1), jnp.float32)
    return jnp.concatenate([zeros, buf[:POST_NMS_TOPN, :4]], axis=1)
